# baseline (speedup 1.0000x reference)
import functools

import jax
import jax.numpy as jnp
from jax import lax
from jax.experimental import pallas as pl
from jax.experimental.pallas import tpu as pltpu
from jax.experimental.pallas import tpu_sc as plsc

CHANNEL = 128
SIZE0, SIZE1 = 4, 4
EMB_DIM = CHANNEL * SIZE0 * SIZE1
BATCH = 16384
VOCAB = 10
VOCAB_PAD = 16
NC, NS = 2, 16
NW = NC * NS
NSEM = 4
GROUP = 16

N_SC = 0
N_TC = BATCH - N_SC
TCB = 512
SC_B_PER_W = max(N_SC // NW, 16)


_mesh = plsc.VectorSubcoreMesh(core_axis_name="c", subcore_axis_name="s")


@functools.partial(
    pl.kernel,
    out_type=jax.ShapeDtypeStruct((BATCH, EMB_DIM), jnp.float32),
    mesh=_mesh,
    scratch_types=[
        pltpu.VMEM((SC_B_PER_W,), jnp.int32),
        pltpu.VMEM((VOCAB, EMB_DIM), jnp.float32),
        pltpu.SemaphoreType.DMA,
        pltpu.SemaphoreType.DMA,
        pltpu.SemaphoreType.DMA,
        pltpu.SemaphoreType.DMA,
    ],
)
def _sc_lookup(digits_hbm, table_hbm, out_hbm, idx_v, table_v,
               s0, s1, s2, s3):
    wid = lax.axis_index("s") * NC + lax.axis_index("c")
    base = wid * SC_B_PER_W
    sems = (s0, s1, s2, s3)

    pltpu.sync_copy(digits_hbm.at[pl.ds(base, SC_B_PER_W)], idx_v)
    pltpu.sync_copy(table_hbm, table_v)

    def fire(g, carry):
        goff = g * GROUP
        vec = idx_v[pl.ds(goff, GROUP)]
        for k in range(GROUP):
            row = vec[k]
            pltpu.async_copy(
                table_v.at[pl.ds(row, 1)],
                out_hbm.at[pl.ds(base + goff + k, 1)],
                sems[k % NSEM],
            )
        return carry

    lax.fori_loop(0, SC_B_PER_W // GROUP, fire, 0)

    def drain(j, carry):
        for p in range(NSEM):
            pltpu.make_async_copy(
                table_hbm.at[pl.ds(0, 8)], table_v.at[pl.ds(0, 8)], sems[p]
            ).wait()
        return carry

    lax.fori_loop(0, SC_B_PER_W // NSEM // 8, drain, 0)


def _tc_body(d_ref, t_ref, full_ref, o_ref):
    del full_ref
    d = d_ref[0]
    onehot = (
        lax.broadcasted_iota(jnp.int32, (TCB, VOCAB_PAD), 1) == d
    ).astype(jnp.float32)
    o_ref[...] = jnp.dot(onehot, t_ref[...],
                         preferred_element_type=jnp.float32)


def _tc_fill(digits3, table_pad, out_full):
    return pl.pallas_call(
        _tc_body,
        grid=(N_TC // TCB,),
        in_specs=[
            pl.BlockSpec((1, TCB, 1), lambda i: (i, 0, 0)),
            pl.BlockSpec((VOCAB_PAD, EMB_DIM), lambda i: (0, 0)),
            pl.BlockSpec(memory_space=pl.ANY),
        ],
        out_specs=pl.BlockSpec((TCB, EMB_DIM),
                               lambda i: (i + N_SC // TCB, 0)),
        out_shape=jax.ShapeDtypeStruct((BATCH, EMB_DIM), jnp.float32),
        input_output_aliases={2: 0},
    )(digits3, table_pad, out_full)


def kernel(digits, table):
    out_full = jnp.zeros((BATCH, EMB_DIM), jnp.float32)
    digits3 = digits[N_SC:].reshape(N_TC // TCB, TCB, 1)
    table_pad = jnp.concatenate(
        [table, jnp.zeros((VOCAB_PAD - VOCAB, EMB_DIM), table.dtype)], axis=0
    )
    out = _tc_fill(digits3, table_pad, out_full)
    return out.reshape(-1, CHANNEL, SIZE0, SIZE1)


# speedup vs baseline: 1.1274x; 1.1274x over previous
import functools

import jax
import jax.numpy as jnp
from jax import lax
from jax.experimental import pallas as pl
from jax.experimental.pallas import tpu as pltpu
from jax.experimental.pallas import tpu_sc as plsc

CHANNEL = 128
SIZE0, SIZE1 = 4, 4
EMB_DIM = CHANNEL * SIZE0 * SIZE1
BATCH = 16384
VOCAB = 10
NC, NS = 2, 16
NW = NC * NS
B_PER_W = BATCH // NW
NSEM = 4
GROUP = 16
BURST = 128
NBURST = B_PER_W // BURST


_mesh = plsc.VectorSubcoreMesh(core_axis_name="c", subcore_axis_name="s")


@functools.partial(
    pl.kernel,
    out_type=jax.ShapeDtypeStruct((BATCH, EMB_DIM), jnp.float32),
    mesh=_mesh,
    scratch_types=[
        pltpu.VMEM((B_PER_W,), jnp.int32),
        pltpu.VMEM((VOCAB, EMB_DIM), jnp.float32),
        pltpu.SemaphoreType.DMA,
        pltpu.SemaphoreType.DMA,
        pltpu.SemaphoreType.DMA,
        pltpu.SemaphoreType.DMA,
    ],
)
def _lookup(digits_hbm, table_hbm, out_hbm, idx_v, table_v, s0, s1, s2, s3):
    wid = lax.axis_index("s") * NC + lax.axis_index("c")
    base = wid * B_PER_W
    sems = (s0, s1, s2, s3)

    pltpu.sync_copy(digits_hbm.at[pl.ds(base, B_PER_W)], idx_v)
    pltpu.sync_copy(table_hbm, table_v)

    def burst(t, carry):
        toff = t * BURST

        def fire(g, c):
            goff = toff + g * GROUP
            vec = idx_v[pl.ds(goff, GROUP)]
            for k in range(GROUP):
                row = vec[k]
                pltpu.async_copy(
                    table_v.at[pl.ds(row, 1)],
                    out_hbm.at[pl.ds(base + goff + k, 1)],
                    sems[k % NSEM],
                )
            return c

        lax.fori_loop(0, BURST // GROUP, fire, 0)

        def drain(j, c):
            for p in range(NSEM):
                pltpu.make_async_copy(
                    table_hbm.at[pl.ds(0, 8)], table_v.at[pl.ds(0, 8)],
                    sems[p],
                ).wait()
            return c

        lax.fori_loop(0, BURST // NSEM // 8, drain, 0)
        return carry

    lax.fori_loop(0, NBURST, burst, 0)


def kernel(digits, table):
    out = _lookup(digits, table)
    return out.reshape(-1, CHANNEL, SIZE0, SIZE1)
